# baseline (speedup 1.0000x reference)
import jax
import jax.numpy as jnp
from jax import lax
from jax.experimental import pallas as pl
from jax.experimental.pallas import tpu as pltpu
from jax.experimental.pallas import tpu_sc as plsc

N = 10000
E = 160000
D = 256
H = 128
NPD = 10240
NC = 2
NS = 16
EPT = E // NS
EB = 80
NCHUNK = EPT // EB
DEG_EPT = E // (NC * NS)
DEG_B = 1000
DEG_NCHUNK = DEG_EPT // DEG_B
NS_EPT = E // (NC * NS)
NS_B = 1000
NS_NCHUNK = NS_EPT // NS_B
DRPT = NPD // NS


def _sc_mesh():
    return plsc.VectorSubcoreMesh(core_axis_name="c", subcore_axis_name="s",
                                  num_cores=NC, num_subcores=NS)


def _deg_body(src_h, pdeg_h, ones_v, idx_v, z_v, shared_deg):
    c = lax.axis_index("c")
    s = lax.axis_index("s")

    def zfill(i, _):
        z_v[pl.ds(i * 16, 16)] = jnp.zeros((16,), jnp.float32)
        return 0

    lax.fori_loop(0, DRPT // 16, zfill, 0)

    def ofill(i, _):
        ones_v[pl.ds(i * 16, 16)] = jnp.ones((16,), jnp.float32)
        return 0

    lax.fori_loop(0, 63, ofill, 0)

    pltpu.sync_copy(z_v, shared_deg.at[pl.ds(s * DRPT, DRPT)])
    plsc.subcore_barrier()

    def chunk(k, _):
        eb = pl.multiple_of(c * (E // 2) + s * DEG_EPT + k * DEG_B, 8)
        pltpu.sync_copy(src_h.at[pl.ds(eb, DEG_B)], idx_v)
        pltpu.sync_copy(ones_v.at[pl.ds(0, DEG_B)], shared_deg.at[idx_v],
                        add=True)
        return 0

    lax.fori_loop(0, DEG_NCHUNK, chunk, 0)
    plsc.subcore_barrier()

    @pl.when(s == 0)
    def _():
        pltpu.sync_copy(shared_deg, pdeg_h.at[c])


def _deg_kernel(src):
    return pl.kernel(
        _deg_body,
        out_type=jax.ShapeDtypeStruct((NC, NPD), jnp.float32),
        mesh=_sc_mesh(),
        scratch_types=[
            pltpu.VMEM((1008,), jnp.float32),
            pltpu.VMEM((DEG_B,), jnp.int32),
            pltpu.VMEM((DRPT,), jnp.float32),
            pltpu.VMEM_SHARED((NPD,), jnp.float32),
        ],
        compiler_params=pltpu.CompilerParams(needs_layout_passes=False),
    )(src)


def _nsrc_body(src_h, norm_h, ns_h, idx_v, val_v):
    c = lax.axis_index("c")
    s = lax.axis_index("s")

    def chunk(k, _):
        eb = pl.multiple_of(c * (E // 2) + s * NS_EPT + k * NS_B, 8)
        pltpu.sync_copy(src_h.at[pl.ds(eb, NS_B)], idx_v)
        pltpu.sync_copy(norm_h.at[idx_v], val_v)
        pltpu.sync_copy(val_v, ns_h.at[pl.ds(eb, NS_B)])
        return 0

    lax.fori_loop(0, NS_NCHUNK, chunk, 0)


def _nsrc_kernel(src, norm):
    return pl.kernel(
        _nsrc_body,
        out_type=jax.ShapeDtypeStruct((E,), jnp.float32),
        mesh=_sc_mesh(),
        scratch_types=[
            pltpu.VMEM((NS_B,), jnp.int32),
            pltpu.VMEM((NS_B,), jnp.float32),
        ],
        compiler_params=pltpu.CompilerParams(needs_layout_passes=False),
    )(src, norm)


def _edge_body(xlh_h, ehh_h, rh_h, src_h, dst_h, out_h,
               sidx0, sidx1, didx0, didx1, didx_s0, didx_s1,
               rows0, rows1, ehv0, ehv1,
               sg0, sg1, se0, se1,
               si0, si1, di0, di1, ss0, ss1, shared_out):
    c = lax.axis_index("c")
    s = lax.axis_index("s")

    rb = pl.multiple_of(s * 640, 8)

    @pl.when(s < NS - 1)
    def _():
        pltpu.sync_copy(rh_h.at[c, pl.ds(rb, 640)],
                        shared_out.at[pl.ds(rb, 640)])

    @pl.when(s == NS - 1)
    def _():
        pltpu.sync_copy(rh_h.at[c, pl.ds(rb, N - 640 * (NS - 1))],
                        shared_out.at[pl.ds(rb, N - 640 * (NS - 1))])

    plsc.subcore_barrier()

    bufs = ((sidx0, didx0, didx_s0, rows0, ehv0, sg0, se0, si0, di0, ss0),
            (sidx1, didx1, didx_s1, rows1, ehv1, sg1, se1, si1, di1, ss1))

    def issue_idx(k, p):
        sidx, didx = bufs[p][0], bufs[p][1]
        si, di = bufs[p][7], bufs[p][8]
        eb = pl.multiple_of(s * EPT + k * EB, 8)
        pltpu.async_copy(src_h.at[pl.ds(eb, EB)], sidx, si)
        pltpu.async_copy(dst_h.at[pl.ds(eb, EB)], didx, di)

    def wait_idx(p):
        sidx, didx = bufs[p][0], bufs[p][1]
        si, di = bufs[p][7], bufs[p][8]
        eb = pl.multiple_of(s * EPT, 8)
        pltpu.make_async_copy(src_h.at[pl.ds(eb, EB)], sidx, si).wait()
        pltpu.make_async_copy(dst_h.at[pl.ds(eb, EB)], didx, di).wait()

    def issue_gathers(p):
        sidx, rows, sg = bufs[p][0], bufs[p][3], bufs[p][5]
        pltpu.async_copy(xlh_h.at[c].at[sidx], rows, sg)

    def wait_gathers(p):
        sidx, rows, sg = bufs[p][0], bufs[p][3], bufs[p][5]
        pltpu.make_async_copy(xlh_h.at[c].at[sidx], rows, sg).wait()

    def issue_eh(k, p):
        ehv, se = bufs[p][4], bufs[p][6]
        eb = pl.multiple_of(s * EPT + k * EB, 8)
        pltpu.async_copy(ehh_h.at[c, pl.ds(eb, EB)], ehv, se)

    def wait_eh(p):
        ehv, se = bufs[p][4], bufs[p][6]
        eb = pl.multiple_of(s * EPT, 8)
        pltpu.make_async_copy(ehh_h.at[c, pl.ds(eb, EB)], ehv, se).wait()

    def issue_scatter(p):
        didx_s, ehv, ss = bufs[p][2], bufs[p][4], bufs[p][9]
        pltpu.async_copy(ehv, shared_out.at[didx_s], ss, add=True)

    def wait_scatter(p):
        didx_s, ehv, ss = bufs[p][2], bufs[p][4], bufs[p][9]
        pltpu.make_async_copy(ehv, shared_out.at[didx_s], ss).wait()

    def phase(k, p):
        didx, didx_s = bufs[p][1], bufs[p][2]
        rows, ehv = bufs[p][3], bufs[p][4]

        @pl.when(k + 1 <= NCHUNK - 1)
        def _():
            wait_idx(1 - p)
            issue_gathers(1 - p)

        @pl.when(k > 0)
        def _():
            wait_scatter(1 - p)

        @pl.when(k + 1 <= NCHUNK - 1)
        def _():
            issue_eh(k + 1, 1 - p)

        wait_gathers(p)
        wait_eh(p)

        def nloop(i, _):
            sl = pl.ds(i * 16, 16)
            didx_s[sl] = didx[sl]
            return 0

        lax.fori_loop(0, EB // 16, nloop, 0)

        @pl.when(k + 2 <= NCHUNK - 1)
        def _():
            issue_idx(k + 2, p)

        def eg(g, _):
            for e16 in range(16):
                e = g * 16 + e16
                for j in range(H // 16):
                    sl = pl.ds(j * 16, 16)
                    ehv[e, sl] = jnp.maximum(rows[e, sl] + ehv[e, sl], 0.0)
            return 0

        lax.fori_loop(0, EB // 16, eg, 0)
        issue_scatter(p)

    issue_idx(0, 0)
    issue_idx(1, 1)
    wait_idx(0)
    issue_gathers(0)
    issue_eh(0, 0)

    def pair(i, _):
        phase(2 * i, 0)
        phase(2 * i + 1, 1)
        return 0

    lax.fori_loop(0, (NCHUNK - 1) // 2, pair, 0)
    phase(NCHUNK - 1, 0)
    wait_scatter(0)

    plsc.subcore_barrier()

    @pl.when(s < NS - 1)
    def _():
        pltpu.sync_copy(shared_out.at[pl.ds(rb, 640)],
                        out_h.at[c, pl.ds(rb, 640)])

    @pl.when(s == NS - 1)
    def _():
        pltpu.sync_copy(shared_out.at[pl.ds(rb, N - 640 * (NS - 1))],
                        out_h.at[c, pl.ds(rb, N - 640 * (NS - 1))])


def _edge_kernel(xlh, ehh, rh, src, dst):
    return pl.kernel(
        _edge_body,
        out_type=jax.ShapeDtypeStruct((NC, N, H), jnp.float32),
        mesh=_sc_mesh(),
        scratch_types=[
            pltpu.VMEM((EB,), jnp.int32),
            pltpu.VMEM((EB,), jnp.int32),
            pltpu.VMEM((EB,), jnp.int32),
            pltpu.VMEM((EB,), jnp.int32),
            pltpu.VMEM((EB,), jnp.int32),
            pltpu.VMEM((EB,), jnp.int32),
            pltpu.VMEM((EB, H), jnp.float32),
            pltpu.VMEM((EB, H), jnp.float32),
            pltpu.VMEM((EB, H), jnp.float32),
            pltpu.VMEM((EB, H), jnp.float32),
            pltpu.SemaphoreType.DMA,
            pltpu.SemaphoreType.DMA,
            pltpu.SemaphoreType.DMA,
            pltpu.SemaphoreType.DMA,
            pltpu.SemaphoreType.DMA,
            pltpu.SemaphoreType.DMA,
            pltpu.SemaphoreType.DMA,
            pltpu.SemaphoreType.DMA,
            pltpu.SemaphoreType.DMA,
            pltpu.SemaphoreType.DMA,
            pltpu.VMEM_SHARED((N, H), jnp.float32),
        ],
        compiler_params=pltpu.CompilerParams(needs_layout_passes=False),
    )(xlh, ehh, rh, src, dst)


def _norm_body(pdeg_ref, norm_ref):
    d = pdeg_ref[0] + pdeg_ref[1] + 1.0
    norm_ref[...] = lax.rsqrt(d)


def _norm_kernel(pdeg):
    pdeg2 = pdeg.reshape(NC, NPD // 128, 128)
    norm = pl.pallas_call(
        _norm_body,
        out_shape=jax.ShapeDtypeStruct((NPD // 128, 128), jnp.float32),
    )(pdeg2)
    return norm.reshape(NPD)


def _dense_body(x_ref, w_ref, b_ref, re_ref, nrm_ref, xlh_ref, rh_ref):
    xl = jnp.dot(x_ref[...], w_ref[...],
                 preferred_element_type=jnp.float32) + b_ref[...]
    nrm = nrm_ref[...]
    xln = xl * nrm
    rn = jnp.maximum(xl + re_ref[...], 0.0) * nrm
    xlh_ref[0] = xln[:, :H]
    xlh_ref[1] = xln[:, H:]
    rh_ref[0] = rn[:, :H]
    rh_ref[1] = rn[:, H:]


def _dense_kernel(x, W_lin, b_lin, root_emb, norm_col):
    blk = 1000
    grid = N // blk
    return pl.pallas_call(
        _dense_body,
        grid=(grid,),
        in_specs=[
            pl.BlockSpec((blk, D), lambda j: (j, 0)),
            pl.BlockSpec((D, D), lambda j: (0, 0)),
            pl.BlockSpec((1, D), lambda j: (0, 0)),
            pl.BlockSpec((1, D), lambda j: (0, 0)),
            pl.BlockSpec((blk, 1), lambda j: (j, 0)),
        ],
        out_specs=[
            pl.BlockSpec((NC, blk, H), lambda j: (0, j, 0)),
            pl.BlockSpec((NC, blk, H), lambda j: (0, j, 0)),
        ],
        out_shape=[
            jax.ShapeDtypeStruct((NC, N, H), jnp.float32),
            jax.ShapeDtypeStruct((NC, N, H), jnp.float32),
        ],
    )(x, W_lin, b_lin.reshape(1, D), root_emb.reshape(1, D), norm_col)


def _eh_body(ex_ref, w_ref, b_ref, ns_ref, ehh_ref):
    eh = (jnp.dot(ex_ref[...], w_ref[...],
                  preferred_element_type=jnp.float32)
          + b_ref[...]) * ns_ref[...]
    ehh_ref[0] = eh[:, :H]
    ehh_ref[1] = eh[:, H:]


def _eh_kernel(ex_pad, W_edge_pad, b_edge, ns_col):
    blk = 2000
    grid = E // blk
    return pl.pallas_call(
        _eh_body,
        grid=(grid,),
        in_specs=[
            pl.BlockSpec((blk, 8), lambda j: (j, 0)),
            pl.BlockSpec((8, D), lambda j: (0, 0)),
            pl.BlockSpec((1, D), lambda j: (0, 0)),
            pl.BlockSpec((blk, 1), lambda j: (j, 0)),
        ],
        out_specs=pl.BlockSpec((NC, blk, H), lambda j: (0, j, 0)),
        out_shape=jax.ShapeDtypeStruct((NC, E, H), jnp.float32),
    )(ex_pad, W_edge_pad, b_edge.reshape(1, D), ns_col)


def _finish_body(acc_ref, nrm_ref, out_ref):
    nrm = nrm_ref[...]
    out_ref[:, :H] = acc_ref[0] * nrm
    out_ref[:, H:] = acc_ref[1] * nrm


def _finish_kernel(acc, norm_col):
    blk = 1000
    grid = N // blk
    return pl.pallas_call(
        _finish_body,
        grid=(grid,),
        in_specs=[
            pl.BlockSpec((NC, blk, H), lambda j: (0, j, 0)),
            pl.BlockSpec((blk, 1), lambda j: (j, 0)),
        ],
        out_specs=pl.BlockSpec((blk, D), lambda j: (j, 0)),
        out_shape=jax.ShapeDtypeStruct((N, D), jnp.float32),
    )(acc, norm_col)


@jax.jit
def kernel(x, edge_index, ex, W_lin, b_lin, W_edge, b_edge, root_emb):
    src = edge_index[0]
    dst = edge_index[1]

    ex_pad = jnp.pad(ex, ((0, 0), (0, 1)))
    W_edge_pad = jnp.pad(W_edge, ((0, 1), (0, 0)))

    pdeg = _deg_kernel(src)
    norm = _norm_kernel(pdeg)
    norm_col = norm[:N].reshape(N, 1)
    ns_e = _nsrc_kernel(src, norm)
    xlh, rh = _dense_kernel(x, W_lin, b_lin, root_emb, norm_col)
    ehh = _eh_kernel(ex_pad, W_edge_pad, b_edge, ns_e.reshape(E, 1))
    acc = _edge_kernel(xlh, ehh, rh, src, dst)
    return _finish_kernel(acc, norm_col)


# speedup vs baseline: 1.0210x; 1.0210x over previous
import jax
import jax.numpy as jnp
from jax import lax
from jax.experimental import pallas as pl
from jax.experimental.pallas import tpu as pltpu
from jax.experimental.pallas import tpu_sc as plsc

N = 10000
E = 160000
D = 256
H = 128
NPD = 10240
NC = 2
NS = 16
EPT = E // NS
EB = 80
NCHUNK = EPT // EB
DEG_B = 1000
DEG_NCHUNK = EPT // DEG_B
NS_EPT = E // (NC * NS)
NS_B = 1000
NS_NCHUNK = NS_EPT // NS_B
DRPT = NPD // NS


def _sc_mesh():
    return plsc.VectorSubcoreMesh(core_axis_name="c", subcore_axis_name="s",
                                  num_cores=NC, num_subcores=NS)


def _front_body(src_h, norm_h, ns_h, ones_v, idx_v, dv, nv, shared_deg):
    c = lax.axis_index("c")
    s = lax.axis_index("s")

    def zfill(i, _):
        dv[pl.ds(i * 16, 16)] = jnp.zeros((16,), jnp.float32)
        return 0

    lax.fori_loop(0, DRPT // 16, zfill, 0)

    def ofill(i, _):
        ones_v[pl.ds(i * 16, 16)] = jnp.ones((16,), jnp.float32)
        return 0

    lax.fori_loop(0, 63, ofill, 0)

    pltpu.sync_copy(dv, shared_deg.at[pl.ds(s * DRPT, DRPT)])
    plsc.subcore_barrier()

    def chunk(k, _):
        eb = pl.multiple_of(s * EPT + k * DEG_B, 8)
        pltpu.sync_copy(src_h.at[pl.ds(eb, DEG_B)], idx_v)
        pltpu.sync_copy(ones_v.at[pl.ds(0, DEG_B)], shared_deg.at[idx_v],
                        add=True)
        return 0

    lax.fori_loop(0, DEG_NCHUNK, chunk, 0)
    plsc.subcore_barrier()

    rb = pl.multiple_of(s * DRPT, 8)
    pltpu.sync_copy(shared_deg.at[pl.ds(rb, DRPT)], dv)

    def rsq(i, _):
        sl = pl.ds(i * 16, 16)
        d = dv[sl] + 1.0
        bits = lax.bitcast_convert_type(d, jnp.int32)
        seed = 0x5F3759DF - lax.shift_right_arithmetic(bits, 1)
        y = lax.bitcast_convert_type(seed, jnp.float32)
        y = y * (1.5 - 0.5 * d * y * y)
        y = y * (1.5 - 0.5 * d * y * y)
        y = y * (1.5 - 0.5 * d * y * y)
        nv[sl] = y
        return 0

    lax.fori_loop(0, DRPT // 16, rsq, 0)
    pltpu.sync_copy(nv, shared_deg.at[pl.ds(rb, DRPT)])

    @pl.when(c == 0)
    def _():
        pltpu.sync_copy(nv, norm_h.at[pl.ds(rb, DRPT)])

    plsc.subcore_barrier()

    def nchunk(k, _):
        eb = pl.multiple_of(c * (E // 2) + s * NS_EPT + k * NS_B, 8)
        pltpu.sync_copy(src_h.at[pl.ds(eb, NS_B)], idx_v)
        pltpu.sync_copy(shared_deg.at[idx_v], ones_v.at[pl.ds(0, NS_B)])
        pltpu.sync_copy(ones_v.at[pl.ds(0, NS_B)], ns_h.at[pl.ds(eb, NS_B)])
        return 0

    lax.fori_loop(0, NS_NCHUNK, nchunk, 0)


def _front_kernel(src):
    return pl.kernel(
        _front_body,
        out_type=[
            jax.ShapeDtypeStruct((NPD,), jnp.float32),
            jax.ShapeDtypeStruct((E,), jnp.float32),
        ],
        mesh=_sc_mesh(),
        scratch_types=[
            pltpu.VMEM((1008,), jnp.float32),
            pltpu.VMEM((DEG_B,), jnp.int32),
            pltpu.VMEM((DRPT,), jnp.float32),
            pltpu.VMEM((DRPT,), jnp.float32),
            pltpu.VMEM_SHARED((NPD,), jnp.float32),
        ],
        compiler_params=pltpu.CompilerParams(needs_layout_passes=False),
    )(src)


def _edge_body(xlh_h, ehh_h, rh_h, src_h, dst_h, out_h,
               sidx0, sidx1, didx0, didx1, didx_s0, didx_s1,
               rows0, rows1, ehv0, ehv1,
               sg0, sg1, se0, se1,
               si0, si1, di0, di1, ss0, ss1, shared_out):
    c = lax.axis_index("c")
    s = lax.axis_index("s")

    rb = pl.multiple_of(s * 640, 8)

    @pl.when(s < NS - 1)
    def _():
        pltpu.sync_copy(rh_h.at[c, pl.ds(rb, 640)],
                        shared_out.at[pl.ds(rb, 640)])

    @pl.when(s == NS - 1)
    def _():
        pltpu.sync_copy(rh_h.at[c, pl.ds(rb, N - 640 * (NS - 1))],
                        shared_out.at[pl.ds(rb, N - 640 * (NS - 1))])

    plsc.subcore_barrier()

    bufs = ((sidx0, didx0, didx_s0, rows0, ehv0, sg0, se0, si0, di0, ss0),
            (sidx1, didx1, didx_s1, rows1, ehv1, sg1, se1, si1, di1, ss1))

    def issue_idx(k, p):
        sidx, didx = bufs[p][0], bufs[p][1]
        si, di = bufs[p][7], bufs[p][8]
        eb = pl.multiple_of(s * EPT + k * EB, 8)
        pltpu.async_copy(src_h.at[pl.ds(eb, EB)], sidx, si)
        pltpu.async_copy(dst_h.at[pl.ds(eb, EB)], didx, di)

    def wait_idx(p):
        sidx, didx = bufs[p][0], bufs[p][1]
        si, di = bufs[p][7], bufs[p][8]
        eb = pl.multiple_of(s * EPT, 8)
        pltpu.make_async_copy(src_h.at[pl.ds(eb, EB)], sidx, si).wait()
        pltpu.make_async_copy(dst_h.at[pl.ds(eb, EB)], didx, di).wait()

    def issue_gathers(p):
        sidx, rows, sg = bufs[p][0], bufs[p][3], bufs[p][5]
        pltpu.async_copy(xlh_h.at[c].at[sidx], rows, sg)

    def wait_gathers(p):
        sidx, rows, sg = bufs[p][0], bufs[p][3], bufs[p][5]
        pltpu.make_async_copy(xlh_h.at[c].at[sidx], rows, sg).wait()

    def issue_eh(k, p):
        ehv, se = bufs[p][4], bufs[p][6]
        eb = pl.multiple_of(s * EPT + k * EB, 8)
        pltpu.async_copy(ehh_h.at[c, pl.ds(eb, EB)], ehv, se)

    def wait_eh(p):
        ehv, se = bufs[p][4], bufs[p][6]
        eb = pl.multiple_of(s * EPT, 8)
        pltpu.make_async_copy(ehh_h.at[c, pl.ds(eb, EB)], ehv, se).wait()

    def issue_scatter(p):
        didx_s, ehv, ss = bufs[p][2], bufs[p][4], bufs[p][9]
        pltpu.async_copy(ehv, shared_out.at[didx_s], ss, add=True)

    def wait_scatter(p):
        didx_s, ehv, ss = bufs[p][2], bufs[p][4], bufs[p][9]
        pltpu.make_async_copy(ehv, shared_out.at[didx_s], ss).wait()

    def phase(k, p):
        didx, didx_s = bufs[p][1], bufs[p][2]
        rows, ehv = bufs[p][3], bufs[p][4]

        @pl.when(k + 1 <= NCHUNK - 1)
        def _():
            wait_idx(1 - p)
            issue_gathers(1 - p)

        @pl.when(k > 0)
        def _():
            wait_scatter(1 - p)

        @pl.when(k + 1 <= NCHUNK - 1)
        def _():
            issue_eh(k + 1, 1 - p)

        wait_gathers(p)
        wait_eh(p)

        def nloop(i, _):
            sl = pl.ds(i * 16, 16)
            didx_s[sl] = didx[sl]
            return 0

        lax.fori_loop(0, EB // 16, nloop, 0)

        @pl.when(k + 2 <= NCHUNK - 1)
        def _():
            issue_idx(k + 2, p)

        def eg(g, _):
            for e16 in range(16):
                e = g * 16 + e16
                for j in range(H // 16):
                    sl = pl.ds(j * 16, 16)
                    ehv[e, sl] = jnp.maximum(rows[e, sl] + ehv[e, sl], 0.0)
            return 0

        lax.fori_loop(0, EB // 16, eg, 0)
        issue_scatter(p)

    issue_idx(0, 0)
    issue_idx(1, 1)
    wait_idx(0)
    issue_gathers(0)
    issue_eh(0, 0)

    if NCHUNK % 2 == 1:
        def pair(i, _):
            phase(2 * i, 0)
            phase(2 * i + 1, 1)
            return 0

        lax.fori_loop(0, (NCHUNK - 1) // 2, pair, 0)
        phase(NCHUNK - 1, 0)
        wait_scatter(0)
    else:
        def pair(i, _):
            phase(2 * i, 0)
            phase(2 * i + 1, 1)
            return 0

        lax.fori_loop(0, (NCHUNK - 2) // 2, pair, 0)
        phase(NCHUNK - 2, 0)
        phase(NCHUNK - 1, 1)
        wait_scatter(1)

    plsc.subcore_barrier()

    @pl.when(s < NS - 1)
    def _():
        pltpu.sync_copy(shared_out.at[pl.ds(rb, 640)],
                        out_h.at[c, pl.ds(rb, 640)])

    @pl.when(s == NS - 1)
    def _():
        pltpu.sync_copy(shared_out.at[pl.ds(rb, N - 640 * (NS - 1))],
                        out_h.at[c, pl.ds(rb, N - 640 * (NS - 1))])


def _edge_kernel(xlh, ehh, rh, src, dst):
    return pl.kernel(
        _edge_body,
        out_type=jax.ShapeDtypeStruct((NC, N, H), jnp.float32),
        mesh=_sc_mesh(),
        scratch_types=[
            pltpu.VMEM((EB,), jnp.int32),
            pltpu.VMEM((EB,), jnp.int32),
            pltpu.VMEM((EB,), jnp.int32),
            pltpu.VMEM((EB,), jnp.int32),
            pltpu.VMEM((EB,), jnp.int32),
            pltpu.VMEM((EB,), jnp.int32),
            pltpu.VMEM((EB, H), jnp.float32),
            pltpu.VMEM((EB, H), jnp.float32),
            pltpu.VMEM((EB, H), jnp.float32),
            pltpu.VMEM((EB, H), jnp.float32),
            pltpu.SemaphoreType.DMA,
            pltpu.SemaphoreType.DMA,
            pltpu.SemaphoreType.DMA,
            pltpu.SemaphoreType.DMA,
            pltpu.SemaphoreType.DMA,
            pltpu.SemaphoreType.DMA,
            pltpu.SemaphoreType.DMA,
            pltpu.SemaphoreType.DMA,
            pltpu.SemaphoreType.DMA,
            pltpu.SemaphoreType.DMA,
            pltpu.VMEM_SHARED((N, H), jnp.float32),
        ],
        compiler_params=pltpu.CompilerParams(needs_layout_passes=False),
    )(xlh, ehh, rh, src, dst)


def _dense_body(x_ref, w_ref, b_ref, re_ref, nrm_ref, xlh_ref, rh_ref):
    xl = jnp.dot(x_ref[...], w_ref[...],
                 preferred_element_type=jnp.float32) + b_ref[...]
    nrm = nrm_ref[...]
    xln = xl * nrm
    rn = jnp.maximum(xl + re_ref[...], 0.0) * nrm
    xlh_ref[0] = xln[:, :H]
    xlh_ref[1] = xln[:, H:]
    rh_ref[0] = rn[:, :H]
    rh_ref[1] = rn[:, H:]


def _dense_kernel(x, W_lin, b_lin, root_emb, norm_col):
    blk = 1000
    grid = N // blk
    return pl.pallas_call(
        _dense_body,
        grid=(grid,),
        in_specs=[
            pl.BlockSpec((blk, D), lambda j: (j, 0)),
            pl.BlockSpec((D, D), lambda j: (0, 0)),
            pl.BlockSpec((1, D), lambda j: (0, 0)),
            pl.BlockSpec((1, D), lambda j: (0, 0)),
            pl.BlockSpec((blk, 1), lambda j: (j, 0)),
        ],
        out_specs=[
            pl.BlockSpec((NC, blk, H), lambda j: (0, j, 0)),
            pl.BlockSpec((NC, blk, H), lambda j: (0, j, 0)),
        ],
        out_shape=[
            jax.ShapeDtypeStruct((NC, N, H), jnp.float32),
            jax.ShapeDtypeStruct((NC, N, H), jnp.float32),
        ],
    )(x, W_lin, b_lin.reshape(1, D), root_emb.reshape(1, D), norm_col)


def _eh_body(ex_ref, w_ref, b_ref, ns_ref, ehh_ref):
    eh = (jnp.dot(ex_ref[...], w_ref[...],
                  preferred_element_type=jnp.float32)
          + b_ref[...]) * ns_ref[...]
    ehh_ref[0] = eh[:, :H]
    ehh_ref[1] = eh[:, H:]


def _eh_kernel(ex_pad, W_edge_pad, b_edge, ns_col):
    blk = 2000
    grid = E // blk
    return pl.pallas_call(
        _eh_body,
        grid=(grid,),
        in_specs=[
            pl.BlockSpec((blk, 8), lambda j: (j, 0)),
            pl.BlockSpec((8, D), lambda j: (0, 0)),
            pl.BlockSpec((1, D), lambda j: (0, 0)),
            pl.BlockSpec((blk, 1), lambda j: (j, 0)),
        ],
        out_specs=pl.BlockSpec((NC, blk, H), lambda j: (0, j, 0)),
        out_shape=jax.ShapeDtypeStruct((NC, E, H), jnp.float32),
    )(ex_pad, W_edge_pad, b_edge.reshape(1, D), ns_col)


def _finish_body(acc_ref, nrm_ref, out_ref):
    nrm = nrm_ref[...]
    out_ref[:, :H] = acc_ref[0] * nrm
    out_ref[:, H:] = acc_ref[1] * nrm


def _finish_kernel(acc, norm_col):
    blk = 1000
    grid = N // blk
    return pl.pallas_call(
        _finish_body,
        grid=(grid,),
        in_specs=[
            pl.BlockSpec((NC, blk, H), lambda j: (0, j, 0)),
            pl.BlockSpec((blk, 1), lambda j: (j, 0)),
        ],
        out_specs=pl.BlockSpec((blk, D), lambda j: (j, 0)),
        out_shape=jax.ShapeDtypeStruct((N, D), jnp.float32),
    )(acc, norm_col)


@jax.jit
def kernel(x, edge_index, ex, W_lin, b_lin, W_edge, b_edge, root_emb):
    src = edge_index[0]
    dst = edge_index[1]

    ex_pad = jnp.pad(ex, ((0, 0), (0, 1)))
    W_edge_pad = jnp.pad(W_edge, ((0, 1), (0, 0)))

    norm, ns_e = _front_kernel(src)
    norm_col = norm[:N].reshape(N, 1)
    xlh, rh = _dense_kernel(x, W_lin, b_lin, root_emb, norm_col)
    ehh = _eh_kernel(ex_pad, W_edge_pad, b_edge, ns_e.reshape(E, 1))
    acc = _edge_kernel(xlh, ehh, rh, src, dst)
    return _finish_kernel(acc, norm_col)


# speedup vs baseline: 1.3066x; 1.2797x over previous
import jax
import jax.numpy as jnp
from jax import lax
from jax.experimental import pallas as pl
from jax.experimental.pallas import tpu as pltpu
from jax.experimental.pallas import tpu_sc as plsc

N = 10000
E = 160000
D = 256
H = 128
NPD = 10240
NC = 2
NS = 16
EPT = E // NS
EB = 80
NCHUNK = EPT // EB
DEG_B = 1000
DEG_NCHUNK = EPT // DEG_B
NS_EPT = E // (NC * NS)
NS_B = 1000
NS_NCHUNK = NS_EPT // NS_B
DRPT = NPD // NS


def _sc_mesh():
    return plsc.VectorSubcoreMesh(core_axis_name="c", subcore_axis_name="s",
                                  num_cores=NC, num_subcores=NS)


def _front_body(src_h, norm_h, ns_h, ones_v, idx_v, dv, nv, shared_deg):
    c = lax.axis_index("c")
    s = lax.axis_index("s")

    def zfill(i, _):
        dv[pl.ds(i * 16, 16)] = jnp.zeros((16,), jnp.float32)
        return 0

    lax.fori_loop(0, DRPT // 16, zfill, 0)

    def ofill(i, _):
        ones_v[pl.ds(i * 16, 16)] = jnp.ones((16,), jnp.float32)
        return 0

    lax.fori_loop(0, 63, ofill, 0)

    pltpu.sync_copy(dv, shared_deg.at[pl.ds(s * DRPT, DRPT)])
    plsc.subcore_barrier()

    def chunk(k, _):
        eb = pl.multiple_of(s * EPT + k * DEG_B, 8)
        pltpu.sync_copy(src_h.at[pl.ds(eb, DEG_B)], idx_v)
        pltpu.sync_copy(ones_v.at[pl.ds(0, DEG_B)], shared_deg.at[idx_v],
                        add=True)
        return 0

    lax.fori_loop(0, DEG_NCHUNK, chunk, 0)
    plsc.subcore_barrier()

    rb = pl.multiple_of(s * DRPT, 8)
    pltpu.sync_copy(shared_deg.at[pl.ds(rb, DRPT)], dv)

    def rsq(i, _):
        sl = pl.ds(i * 16, 16)
        d = dv[sl] + 1.0
        bits = lax.bitcast_convert_type(d, jnp.int32)
        seed = 0x5F3759DF - lax.shift_right_arithmetic(bits, 1)
        y = lax.bitcast_convert_type(seed, jnp.float32)
        y = y * (1.5 - 0.5 * d * y * y)
        y = y * (1.5 - 0.5 * d * y * y)
        y = y * (1.5 - 0.5 * d * y * y)
        nv[sl] = y
        return 0

    lax.fori_loop(0, DRPT // 16, rsq, 0)
    pltpu.sync_copy(nv, shared_deg.at[pl.ds(rb, DRPT)])

    @pl.when(c == 0)
    def _():
        pltpu.sync_copy(nv, norm_h.at[pl.ds(rb, DRPT)])

    plsc.subcore_barrier()

    def nchunk(k, _):
        eb = pl.multiple_of(c * (E // 2) + s * NS_EPT + k * NS_B, 8)
        pltpu.sync_copy(src_h.at[pl.ds(eb, NS_B)], idx_v)
        pltpu.sync_copy(shared_deg.at[idx_v], ones_v.at[pl.ds(0, NS_B)])
        pltpu.sync_copy(ones_v.at[pl.ds(0, NS_B)], ns_h.at[pl.ds(eb, NS_B)])
        return 0

    lax.fori_loop(0, NS_NCHUNK, nchunk, 0)


def _front_kernel(src):
    return pl.kernel(
        _front_body,
        out_type=[
            jax.ShapeDtypeStruct((NPD,), jnp.float32),
            jax.ShapeDtypeStruct((E,), jnp.float32),
        ],
        mesh=_sc_mesh(),
        scratch_types=[
            pltpu.VMEM((1008,), jnp.float32),
            pltpu.VMEM((DEG_B,), jnp.int32),
            pltpu.VMEM((DRPT,), jnp.float32),
            pltpu.VMEM((DRPT,), jnp.float32),
            pltpu.VMEM_SHARED((NPD,), jnp.float32),
        ],
        compiler_params=pltpu.CompilerParams(needs_layout_passes=False),
    )(src)


def _edge_body(xlh_h, ehh_h, rh_h, src_h, dst_h, out_h,
               sidx0, sidx1, didx0, didx1, didx_s0, didx_s1,
               rows0, rows1, ehv0, ehv1,
               sg0, sg1, se0, se1,
               si0, si1, di0, di1, ss0, ss1, shared_out):
    c = lax.axis_index("c")
    s = lax.axis_index("s")

    rb = pl.multiple_of(s * 640, 8)

    @pl.when(s < NS - 1)
    def _():
        pltpu.sync_copy(rh_h.at[c, pl.ds(rb, 640)],
                        shared_out.at[pl.ds(rb, 640)])

    @pl.when(s == NS - 1)
    def _():
        pltpu.sync_copy(rh_h.at[c, pl.ds(rb, N - 640 * (NS - 1))],
                        shared_out.at[pl.ds(rb, N - 640 * (NS - 1))])

    plsc.subcore_barrier()

    bufs = ((sidx0, didx0, didx_s0, rows0, ehv0, sg0, se0, si0, di0, ss0),
            (sidx1, didx1, didx_s1, rows1, ehv1, sg1, se1, si1, di1, ss1))

    def issue_idx(k, p):
        sidx, didx = bufs[p][0], bufs[p][1]
        si, di = bufs[p][7], bufs[p][8]
        eb = pl.multiple_of(s * EPT + k * EB, 8)
        pltpu.async_copy(src_h.at[pl.ds(eb, EB)], sidx, si)
        pltpu.async_copy(dst_h.at[pl.ds(eb, EB)], didx, di)

    def wait_idx(p):
        sidx, didx = bufs[p][0], bufs[p][1]
        si, di = bufs[p][7], bufs[p][8]
        eb = pl.multiple_of(s * EPT, 8)
        pltpu.make_async_copy(src_h.at[pl.ds(eb, EB)], sidx, si).wait()
        pltpu.make_async_copy(dst_h.at[pl.ds(eb, EB)], didx, di).wait()

    def issue_gathers(p):
        sidx, rows, sg = bufs[p][0], bufs[p][3], bufs[p][5]
        pltpu.async_copy(xlh_h.at[c].at[sidx], rows, sg)

    def wait_gathers(p):
        sidx, rows, sg = bufs[p][0], bufs[p][3], bufs[p][5]
        pltpu.make_async_copy(xlh_h.at[c].at[sidx], rows, sg).wait()

    def issue_eh(k, p):
        ehv, se = bufs[p][4], bufs[p][6]
        eb = pl.multiple_of(s * EPT + k * EB, 8)
        pltpu.async_copy(ehh_h.at[pl.ds(eb, EB), pl.ds(c * H, H)], ehv, se)

    def wait_eh(p):
        ehv, se = bufs[p][4], bufs[p][6]
        eb = pl.multiple_of(s * EPT, 8)
        pltpu.make_async_copy(ehh_h.at[pl.ds(eb, EB), pl.ds(c * H, H)],
                              ehv, se).wait()

    def issue_scatter(p):
        didx_s, ehv, ss = bufs[p][2], bufs[p][4], bufs[p][9]
        pltpu.async_copy(ehv, shared_out.at[didx_s], ss, add=True)

    def wait_scatter(p):
        didx_s, ehv, ss = bufs[p][2], bufs[p][4], bufs[p][9]
        pltpu.make_async_copy(ehv, shared_out.at[didx_s], ss).wait()

    def phase(k, p):
        didx, didx_s = bufs[p][1], bufs[p][2]
        rows, ehv = bufs[p][3], bufs[p][4]

        @pl.when(k + 1 <= NCHUNK - 1)
        def _():
            wait_idx(1 - p)
            issue_gathers(1 - p)

        @pl.when(k > 0)
        def _():
            wait_scatter(1 - p)

        @pl.when(k + 1 <= NCHUNK - 1)
        def _():
            issue_eh(k + 1, 1 - p)

        wait_gathers(p)
        wait_eh(p)

        def nloop(i, _):
            sl = pl.ds(i * 16, 16)
            didx_s[sl] = didx[sl]
            return 0

        lax.fori_loop(0, EB // 16, nloop, 0)

        @pl.when(k + 2 <= NCHUNK - 1)
        def _():
            issue_idx(k + 2, p)

        def eg(g, _):
            for e16 in range(16):
                e = g * 16 + e16
                for j in range(H // 16):
                    sl = pl.ds(j * 16, 16)
                    ehv[e, sl] = jnp.maximum(rows[e, sl] + ehv[e, sl], 0.0)
            return 0

        lax.fori_loop(0, EB // 16, eg, 0)
        issue_scatter(p)

    issue_idx(0, 0)
    issue_idx(1, 1)
    wait_idx(0)
    issue_gathers(0)
    issue_eh(0, 0)

    if NCHUNK % 2 == 1:
        def pair(i, _):
            phase(2 * i, 0)
            phase(2 * i + 1, 1)
            return 0

        lax.fori_loop(0, (NCHUNK - 1) // 2, pair, 0)
        phase(NCHUNK - 1, 0)
        wait_scatter(0)
    else:
        def pair(i, _):
            phase(2 * i, 0)
            phase(2 * i + 1, 1)
            return 0

        lax.fori_loop(0, (NCHUNK - 2) // 2, pair, 0)
        phase(NCHUNK - 2, 0)
        phase(NCHUNK - 1, 1)
        wait_scatter(1)

    plsc.subcore_barrier()

    @pl.when(s < NS - 1)
    def _():
        pltpu.sync_copy(shared_out.at[pl.ds(rb, 640)],
                        out_h.at[c, pl.ds(rb, 640)])

    @pl.when(s == NS - 1)
    def _():
        pltpu.sync_copy(shared_out.at[pl.ds(rb, N - 640 * (NS - 1))],
                        out_h.at[c, pl.ds(rb, N - 640 * (NS - 1))])


def _edge_kernel(xlh, ehh, rh, src, dst):
    return pl.kernel(
        _edge_body,
        out_type=jax.ShapeDtypeStruct((NC, N, H), jnp.float32),
        mesh=_sc_mesh(),
        scratch_types=[
            pltpu.VMEM((EB,), jnp.int32),
            pltpu.VMEM((EB,), jnp.int32),
            pltpu.VMEM((EB,), jnp.int32),
            pltpu.VMEM((EB,), jnp.int32),
            pltpu.VMEM((EB,), jnp.int32),
            pltpu.VMEM((EB,), jnp.int32),
            pltpu.VMEM((EB, H), jnp.float32),
            pltpu.VMEM((EB, H), jnp.float32),
            pltpu.VMEM((EB, H), jnp.float32),
            pltpu.VMEM((EB, H), jnp.float32),
            pltpu.SemaphoreType.DMA,
            pltpu.SemaphoreType.DMA,
            pltpu.SemaphoreType.DMA,
            pltpu.SemaphoreType.DMA,
            pltpu.SemaphoreType.DMA,
            pltpu.SemaphoreType.DMA,
            pltpu.SemaphoreType.DMA,
            pltpu.SemaphoreType.DMA,
            pltpu.SemaphoreType.DMA,
            pltpu.SemaphoreType.DMA,
            pltpu.VMEM_SHARED((N, H), jnp.float32),
        ],
        compiler_params=pltpu.CompilerParams(needs_layout_passes=False),
    )(xlh, ehh, rh, src, dst)


def _dense_body(x_ref, w_ref, b_ref, re_ref, nrm_ref, xlh_ref, rh_ref):
    xl = jnp.dot(x_ref[...], w_ref[...],
                 preferred_element_type=jnp.float32) + b_ref[...]
    nrm = nrm_ref[...]
    xln = xl * nrm
    rn = jnp.maximum(xl + re_ref[...], 0.0) * nrm
    xlh_ref[0] = xln[:, :H]
    xlh_ref[1] = xln[:, H:]
    rh_ref[0] = rn[:, :H]
    rh_ref[1] = rn[:, H:]


def _dense_kernel(x, W_lin, b_lin, root_emb, norm_col):
    blk = 1000
    grid = N // blk
    return pl.pallas_call(
        _dense_body,
        grid=(grid,),
        in_specs=[
            pl.BlockSpec((blk, D), lambda j: (j, 0)),
            pl.BlockSpec((D, D), lambda j: (0, 0)),
            pl.BlockSpec((1, D), lambda j: (0, 0)),
            pl.BlockSpec((1, D), lambda j: (0, 0)),
            pl.BlockSpec((blk, 1), lambda j: (j, 0)),
        ],
        out_specs=[
            pl.BlockSpec((NC, blk, H), lambda j: (0, j, 0)),
            pl.BlockSpec((NC, blk, H), lambda j: (0, j, 0)),
        ],
        out_shape=[
            jax.ShapeDtypeStruct((NC, N, H), jnp.float32),
            jax.ShapeDtypeStruct((NC, N, H), jnp.float32),
        ],
    )(x, W_lin, b_lin.reshape(1, D), root_emb.reshape(1, D), norm_col)


def _eh_body(ex_ref, w_ref, b_ref, ns_ref, ehh_ref):
    j = pl.program_id(0)
    ns_row = ns_ref[pl.ds(j, 1), :]
    ns_col = jnp.transpose(ns_row, (1, 0))
    ehh_ref[...] = (jnp.dot(ex_ref[...], w_ref[...],
                            preferred_element_type=jnp.float32)
                    + b_ref[...]) * ns_col


def _eh_kernel(ex, W_edge, b_edge, ns_e):
    blk = 4000
    grid = E // blk
    return pl.pallas_call(
        _eh_body,
        grid=(grid,),
        in_specs=[
            pl.BlockSpec((blk, 7), lambda j: (j, 0)),
            pl.BlockSpec((7, D), lambda j: (0, 0)),
            pl.BlockSpec((1, D), lambda j: (0, 0)),
            pl.BlockSpec((E // blk, blk), lambda j: (0, 0)),
        ],
        out_specs=pl.BlockSpec((blk, D), lambda j: (j, 0)),
        out_shape=jax.ShapeDtypeStruct((E, D), jnp.float32),
    )(ex, W_edge, b_edge.reshape(1, D), ns_e.reshape(E // blk, blk))


def _finish_body(acc_ref, nrm_ref, out_ref):
    nrm = nrm_ref[...]
    out_ref[:, :H] = acc_ref[0] * nrm
    out_ref[:, H:] = acc_ref[1] * nrm


def _finish_kernel(acc, norm_col):
    blk = 1000
    grid = N // blk
    return pl.pallas_call(
        _finish_body,
        grid=(grid,),
        in_specs=[
            pl.BlockSpec((NC, blk, H), lambda j: (0, j, 0)),
            pl.BlockSpec((blk, 1), lambda j: (j, 0)),
        ],
        out_specs=pl.BlockSpec((blk, D), lambda j: (j, 0)),
        out_shape=jax.ShapeDtypeStruct((N, D), jnp.float32),
    )(acc, norm_col)


@jax.jit
def kernel(x, edge_index, ex, W_lin, b_lin, W_edge, b_edge, root_emb):
    src = edge_index[0]
    dst = edge_index[1]

    norm, ns_e = _front_kernel(src)
    norm_col = norm[:N].reshape(N, 1)
    xlh, rh = _dense_kernel(x, W_lin, b_lin, root_emb, norm_col)
    ehh = _eh_kernel(ex, W_edge, b_edge, ns_e)
    acc = _edge_kernel(xlh, ehh, rh, src, dst)
    return _finish_kernel(acc, norm_col)
